# baseline (speedup 1.0000x reference)
import functools

import jax
import jax.numpy as jnp
from jax import lax
from jax.experimental import pallas as pl
from jax.experimental.pallas import tpu as pltpu
from jax.experimental.pallas import tpu_sc as plsc

_D = 128
_info = plsc.get_sparse_core_info()
_NC = _info.num_cores
_NS = _info.num_subcores
_NW = _NC * _NS

_NBUF = 4


def _make_gather(n_tokens: int, chunk: int):
    per_w = n_tokens // _NW
    n_chunks = per_w // chunk
    n_groups = n_chunks // _NBUF
    mesh = plsc.VectorSubcoreMesh(core_axis_name="c", subcore_axis_name="s")

    @functools.partial(
        pl.kernel,
        mesh=mesh,
        out_type=jax.ShapeDtypeStruct((n_tokens, _D), jnp.float32),
        scratch_types=[pltpu.VMEM((per_w,), jnp.int32)]
        + [pltpu.VMEM((chunk, _D), jnp.float32)] * _NBUF
        + [pltpu.SemaphoreType.DMA] * (2 * _NBUF),
    )
    def k(idx_hbm, table_hbm, out_hbm, idx_v, *rest):
        bufs = rest[:_NBUF]
        gsems = rest[_NBUF:2 * _NBUF]
        wsems = rest[2 * _NBUF:]
        wid = lax.axis_index("s") * _NC + lax.axis_index("c")
        base = wid * per_w
        pltpu.sync_copy(idx_hbm.at[pl.ds(base, per_w)], idx_v)

        def start_gather(c, b):
            return pltpu.async_copy(
                table_hbm.at[idx_v.at[pl.ds(c * chunk, chunk)]],
                bufs[b], gsems[b])

        def wait_gather(b):
            pltpu.make_async_copy(
                table_hbm.at[idx_v.at[pl.ds(0, chunk)]],
                bufs[b], gsems[b]).wait()

        def start_write(c, b):
            return pltpu.async_copy(
                bufs[b], out_hbm.at[pl.ds(base + c * chunk, chunk)], wsems[b])

        def wait_write(b):
            pltpu.make_async_copy(
                bufs[b], out_hbm.at[pl.ds(base, chunk)], wsems[b]).wait()

        start_gather(0, 0)
        start_gather(1, 1)

        def body(g, _):
            for b in range(_NBUF):
                i = g * _NBUF + b
                nxt = (b + 2) % _NBUF
                if b >= 2:
                    @pl.when(g < n_groups - 1)
                    def _():
                        wait_write(nxt)
                        start_gather(i + 2, nxt)
                else:
                    @pl.when(g >= 1)
                    def _():
                        wait_write(nxt)
                    start_gather(i + 2, nxt)
                wait_gather(b)
                start_write(i, b)
            return ()

        lax.fori_loop(0, n_groups, body, ())
        for b in range(_NBUF):
            wait_write(b)

    return k


def _transpose_body(rows_ref, out_ref):
    eye = jnp.eye(_D, dtype=jnp.float32)
    for t in range(8):
        blk = rows_ref[pl.ds(t * _D, _D), :]
        blk_t = lax.dot_general(blk, eye, (((0,), (0,)), ((), ())),
                                precision=lax.Precision.HIGHEST,
                                preferred_element_type=jnp.float32)
        out_ref[:, t] = blk_t.reshape(16, 8, _D)


def _make_transpose(seq: int, batch: int):
    return pl.pallas_call(
        _transpose_body,
        grid=(seq,),
        in_specs=[pl.BlockSpec((batch, _D), lambda s: (s, 0))],
        out_specs=pl.BlockSpec((16, 8, 8, _D), lambda s: (s, 0, 0, 0)),
        out_shape=jax.ShapeDtypeStruct((seq * 16, 8, 8, _D), jnp.float32),
    )


def kernel(x, table):
    batch, seq = x.shape
    idx = x.T.reshape(batch * seq).astype(jnp.int32)
    rows = _make_gather(batch * seq, 80)(idx, table)
    outp = _make_transpose(seq, batch)(rows)
    out = (outp.reshape(seq, 8, 2, 8, 8, _D)
           .transpose(3, 5, 0, 1, 2, 4)
           .reshape(batch, seq, 8, 16))
    return out


# speedup vs baseline: 1.1367x; 1.1367x over previous
import functools

import jax
import jax.numpy as jnp
from jax import lax
from jax.experimental import pallas as pl
from jax.experimental.pallas import tpu as pltpu
from jax.experimental.pallas import tpu_sc as plsc

_D = 128
_info = plsc.get_sparse_core_info()
_NC = _info.num_cores
_NS = _info.num_subcores
_NW = _NC * _NS

_NBUF = 4


def _make_gather(n_tokens: int, chunk: int):
    per_w = n_tokens // _NW
    n_chunks = per_w // chunk
    n_groups = n_chunks // _NBUF
    mesh = plsc.VectorSubcoreMesh(core_axis_name="c", subcore_axis_name="s")

    @functools.partial(
        pl.kernel,
        mesh=mesh,
        out_type=jax.ShapeDtypeStruct((n_tokens, _D), jnp.float32),
        scratch_types=[pltpu.VMEM((per_w,), jnp.int32)]
        + [pltpu.VMEM((chunk, _D), jnp.float32)] * _NBUF
        + [pltpu.SemaphoreType.DMA] * (2 * _NBUF),
    )
    def k(idx_hbm, table_hbm, out_hbm, idx_v, *rest):
        bufs = rest[:_NBUF]
        gsems = rest[_NBUF:2 * _NBUF]
        wsems = rest[2 * _NBUF:]
        wid = lax.axis_index("s") * _NC + lax.axis_index("c")
        base = wid * per_w
        pltpu.sync_copy(idx_hbm.at[pl.ds(base, per_w)], idx_v)

        def start_gather(c, b):
            return pltpu.async_copy(
                table_hbm.at[idx_v.at[pl.ds(c * chunk, chunk)]],
                bufs[b], gsems[b])

        def wait_gather(b):
            pltpu.make_async_copy(
                table_hbm.at[idx_v.at[pl.ds(0, chunk)]],
                bufs[b], gsems[b]).wait()

        def start_write(c, b):
            return pltpu.async_copy(
                bufs[b], out_hbm.at[pl.ds(base + c * chunk, chunk)], wsems[b])

        def wait_write(b):
            pltpu.make_async_copy(
                bufs[b], out_hbm.at[pl.ds(base, chunk)], wsems[b]).wait()

        start_gather(0, 0)
        start_gather(1, 1)

        def body(g, _):
            for b in range(_NBUF):
                i = g * _NBUF + b
                nxt = (b + 2) % _NBUF
                if b >= 2:
                    @pl.when(g < n_groups - 1)
                    def _():
                        wait_write(nxt)
                        start_gather(i + 2, nxt)
                else:
                    @pl.when(g >= 1)
                    def _():
                        wait_write(nxt)
                    start_gather(i + 2, nxt)
                wait_gather(b)
                start_write(i, b)
            return ()

        lax.fori_loop(0, n_groups, body, ())
        for b in range(_NBUF):
            wait_write(b)

    return k


def _transpose_body(rows_ref, out_ref):
    eye = jnp.eye(_D, dtype=jnp.float32)
    for t in range(8):
        blk = rows_ref[pl.ds(t * _D, _D), :]
        blk_t = lax.dot_general(blk, eye, (((0,), (0,)), ((), ())),
                                preferred_element_type=jnp.float32)
        out_ref[:, t] = blk_t.reshape(16, 8, _D)


def _make_transpose(seq: int, batch: int):
    return pl.pallas_call(
        _transpose_body,
        grid=(seq,),
        in_specs=[pl.BlockSpec((batch, _D), lambda s: (s, 0))],
        out_specs=pl.BlockSpec((16, 8, 8, _D), lambda s: (s, 0, 0, 0)),
        out_shape=jax.ShapeDtypeStruct((seq * 16, 8, 8, _D), jnp.float32),
    )


def kernel(x, table):
    batch, seq = x.shape
    idx = x.T.reshape(batch * seq).astype(jnp.int32)
    rows = _make_gather(batch * seq, 80)(idx, table)
    outp = _make_transpose(seq, batch)(rows)
    out = (outp.reshape(seq, 8, 2, 8, 8, _D)
           .transpose(3, 5, 0, 1, 2, 4)
           .reshape(batch, seq, 8, 16))
    return out


# speedup vs baseline: 1.1724x; 1.0314x over previous
import functools

import jax
import jax.numpy as jnp
from jax import lax
from jax.experimental import pallas as pl
from jax.experimental.pallas import tpu as pltpu
from jax.experimental.pallas import tpu_sc as plsc

_D = 128
_info = plsc.get_sparse_core_info()
_NC = _info.num_cores
_NS = _info.num_subcores
_NW = _NC * _NS

_NBUF = 4


def _make_gather(n_tokens: int, chunk: int):
    per_w = n_tokens // _NW
    n_chunks = per_w // chunk
    n_groups = n_chunks // _NBUF
    mesh = plsc.VectorSubcoreMesh(core_axis_name="c", subcore_axis_name="s")

    @functools.partial(
        pl.kernel,
        mesh=mesh,
        out_type=jax.ShapeDtypeStruct((n_tokens, _D), jnp.float32),
        scratch_types=[pltpu.VMEM((per_w,), jnp.int32)]
        + [pltpu.VMEM((chunk, _D), jnp.float32)] * _NBUF
        + [pltpu.SemaphoreType.DMA] * (2 * _NBUF),
    )
    def k(idx_hbm, table_hbm, out_hbm, idx_v, *rest):
        bufs = rest[:_NBUF]
        gsems = rest[_NBUF:2 * _NBUF]
        wsems = rest[2 * _NBUF:]
        wid = lax.axis_index("s") * _NC + lax.axis_index("c")
        base = wid * per_w
        pltpu.sync_copy(idx_hbm.at[pl.ds(base, per_w)], idx_v)

        def start_gather(c, b):
            return pltpu.async_copy(
                table_hbm.at[idx_v.at[pl.ds(c * chunk, chunk)]],
                bufs[b], gsems[b])

        def wait_gather(b):
            pltpu.make_async_copy(
                table_hbm.at[idx_v.at[pl.ds(0, chunk)]],
                bufs[b], gsems[b]).wait()

        def start_write(c, b):
            return pltpu.async_copy(
                bufs[b], out_hbm.at[pl.ds(base + c * chunk, chunk)], wsems[b])

        def wait_write(b):
            pltpu.make_async_copy(
                bufs[b], out_hbm.at[pl.ds(base, chunk)], wsems[b]).wait()

        start_gather(0, 0)
        start_gather(1, 1)

        def body(g, _):
            for b in range(_NBUF):
                i = g * _NBUF + b
                nxt = (b + 2) % _NBUF
                if b >= 2:
                    @pl.when(g < n_groups - 1)
                    def _():
                        wait_write(nxt)
                        start_gather(i + 2, nxt)
                else:
                    @pl.when(g >= 1)
                    def _():
                        wait_write(nxt)
                    start_gather(i + 2, nxt)
                wait_gather(b)
                start_write(i, b)
            return ()

        lax.fori_loop(0, n_groups, body, ())
        for b in range(_NBUF):
            wait_write(b)

    return k


def _transpose_body(rows_ref, out_ref):
    for t in range(8):
        blk = rows_ref[pl.ds(t * _D, _D), :]
        out_ref[:, t] = blk.T.reshape(16, 8, _D)


def _make_transpose(seq: int, batch: int):
    return pl.pallas_call(
        _transpose_body,
        grid=(seq,),
        in_specs=[pl.BlockSpec((batch, _D), lambda s: (s, 0))],
        out_specs=pl.BlockSpec((16, 8, 8, _D), lambda s: (s, 0, 0, 0)),
        out_shape=jax.ShapeDtypeStruct((seq * 16, 8, 8, _D), jnp.float32),
    )


def kernel(x, table):
    batch, seq = x.shape
    idx = x.T.reshape(batch * seq).astype(jnp.int32)
    rows = _make_gather(batch * seq, 80)(idx, table)
    outp = _make_transpose(seq, batch)(rows)
    out = (outp.reshape(seq, 8, 2, 8, 8, _D)
           .transpose(3, 5, 0, 1, 2, 4)
           .reshape(batch, seq, 8, 16))
    return out
